# baseline (device time: 49673 ns/iter reference)
import functools

import jax
import jax.numpy as jnp
from jax import lax
from jax.experimental import pallas as pl
from jax.experimental.pallas import tpu as pltpu

N_DEV = 4
SEQ = 1024
HALO = 128
EXT = SEQ + 2 * HALO
HQ = 8
DH = 128
D = HQ * DH
WINDOW = 128
SCALE = 0.08838834764831843


def kernel(x, Wq, K_ext, V_ext, Wo):
    x2 = x.reshape(SEQ, D)
    K2 = K_ext.reshape(SEQ, D)
    V2 = V_ext.reshape(SEQ, D)

    def body(x_ref, wq_ref, k_ref, v_ref, wo_ref, out_ref,
             ext_k, ext_v, ctx_ref, send_sems, recv_sems):
        my = lax.axis_index("i")
        left = (my - 1) % N_DEV
        right = (my + 1) % N_DEV

        barrier = pltpu.get_barrier_semaphore()
        for nbr in (left, right):
            pl.semaphore_signal(barrier, inc=1, device_id=(nbr,),
                                device_id_type=pl.DeviceIdType.MESH)
        pl.semaphore_wait(barrier, 2)

        def send_right(tref, ext_ref, si):
            return pltpu.make_async_remote_copy(
                src_ref=tref.at[pl.ds(SEQ - HALO, HALO)],
                dst_ref=ext_ref.at[pl.ds(0, HALO)],
                send_sem=send_sems.at[si],
                recv_sem=recv_sems.at[si],
                device_id=(right,),
                device_id_type=pl.DeviceIdType.MESH,
            )

        def send_left(tref, ext_ref, si):
            return pltpu.make_async_remote_copy(
                src_ref=tref.at[pl.ds(0, HALO)],
                dst_ref=ext_ref.at[pl.ds(EXT - HALO, HALO)],
                send_sem=send_sems.at[si],
                recv_sem=recv_sems.at[si],
                device_id=(left,),
                device_id_type=pl.DeviceIdType.MESH,
            )

        @pl.when(my < N_DEV - 1)
        def _():
            send_right(k_ref, ext_k, 0).start()
            send_right(v_ref, ext_v, 1).start()

        @pl.when(my > 0)
        def _():
            send_left(k_ref, ext_k, 2).start()
            send_left(v_ref, ext_v, 3).start()

        ext_k[pl.ds(HALO, SEQ), :] = k_ref[:, :]
        ext_v[pl.ds(HALO, SEQ), :] = v_ref[:, :]

        zeros = jnp.zeros((HALO, D), jnp.float32)

        @pl.when(my == 0)
        def _():
            ext_k[pl.ds(0, HALO), :] = zeros
            ext_v[pl.ds(0, HALO), :] = zeros

        @pl.when(my == N_DEV - 1)
        def _():
            ext_k[pl.ds(EXT - HALO, HALO), :] = zeros
            ext_v[pl.ds(EXT - HALO, HALO), :] = zeros

        q = jnp.dot(x_ref[:, :], wq_ref[:, :],
                    preferred_element_type=jnp.float32)

        @pl.when(my > 0)
        def _():
            send_right(k_ref, ext_k, 0).wait_recv()
            send_right(v_ref, ext_v, 1).wait_recv()

        @pl.when(my < N_DEV - 1)
        def _():
            send_left(k_ref, ext_k, 2).wait_recv()
            send_left(v_ref, ext_v, 3).wait_recv()

        @pl.when(my < N_DEV - 1)
        def _():
            send_right(k_ref, ext_k, 0).wait_send()
            send_right(v_ref, ext_v, 1).wait_send()

        @pl.when(my > 0)
        def _():
            send_left(k_ref, ext_k, 2).wait_send()
            send_left(v_ref, ext_v, 3).wait_send()

        qi = lax.broadcasted_iota(jnp.int32, (SEQ, EXT), 0)
        ki = lax.broadcasted_iota(jnp.int32, (SEQ, EXT), 1)
        kg = my * SEQ - HALO + ki
        valid = (jnp.abs(qi + HALO - ki) <= WINDOW) & (kg >= 0) & (kg < N_DEV * SEQ)
        bias = jnp.where(valid, 0.0, -1e9).astype(jnp.float32)

        for h in range(HQ):
            qh = q[:, h * DH:(h + 1) * DH]
            kh = ext_k[:, h * DH:(h + 1) * DH]
            vh = ext_v[:, h * DH:(h + 1) * DH]
            s = lax.dot_general(
                qh, kh, (((1,), (1,)), ((), ())),
                preferred_element_type=jnp.float32,
            ) * SCALE + bias
            m = jnp.max(s, axis=1, keepdims=True)
            w = jnp.exp(s - m)
            w = w / jnp.sum(w, axis=1, keepdims=True)
            ctx_ref[:, h * DH:(h + 1) * DH] = jnp.dot(
                w, vh, preferred_element_type=jnp.float32)

        out_ref[:, :] = jnp.dot(ctx_ref[:, :], wo_ref[:, :],
                                preferred_element_type=jnp.float32)

        @functools.partial(pl.run_scoped, sem2=pltpu.SemaphoreType.REGULAR)
        def _(sem2):
            for nbr in (left, right):
                pl.semaphore_signal(sem2, inc=1, device_id=(nbr,),
                                    device_id_type=pl.DeviceIdType.MESH)
            pl.semaphore_wait(sem2, 2)

    out = pl.pallas_call(
        body,
        out_shape=jax.ShapeDtypeStruct((SEQ, D), jnp.float32),
        in_specs=[pl.BlockSpec(memory_space=pltpu.VMEM)] * 5,
        out_specs=pl.BlockSpec(memory_space=pltpu.VMEM),
        scratch_shapes=[
            pltpu.VMEM((EXT, D), jnp.float32),
            pltpu.VMEM((EXT, D), jnp.float32),
            pltpu.VMEM((SEQ, D), jnp.float32),
            pltpu.SemaphoreType.DMA((4,)),
            pltpu.SemaphoreType.DMA((4,)),
        ],
        compiler_params=pltpu.CompilerParams(collective_id=0),
    )(x2, Wq, K2, V2, Wo)
    return out.reshape(1, SEQ, D)


# device time: 33233 ns/iter; 1.4947x vs baseline; 1.4947x over previous
import functools

import jax
import jax.numpy as jnp
from jax import lax
from jax.experimental import pallas as pl
from jax.experimental.pallas import tpu as pltpu

N_DEV = 4
SEQ = 1024
HALO = 128
EXT = SEQ + 2 * HALO
HQ = 8
DH = 128
D = HQ * DH
WINDOW = 128
SCALE = 0.08838834764831843


def kernel(x, Wq, K_ext, V_ext, Wo):
    x2 = x.reshape(SEQ, D)
    K2 = K_ext.reshape(SEQ, D)
    V2 = V_ext.reshape(SEQ, D)

    def body(x_ref, wq_ref, k_ref, v_ref, wo_ref, out_ref,
             ext_k, ext_v, ctx_ref, send_sems, recv_sems):
        my = lax.axis_index("i")
        left = (my - 1) % N_DEV
        right = (my + 1) % N_DEV

        barrier = pltpu.get_barrier_semaphore()
        for nbr in (left, right):
            pl.semaphore_signal(barrier, inc=1, device_id=(nbr,),
                                device_id_type=pl.DeviceIdType.MESH)
        pl.semaphore_wait(barrier, 2)

        def send_right(tref, ext_ref, si):
            return pltpu.make_async_remote_copy(
                src_ref=tref.at[pl.ds(SEQ - HALO, HALO)],
                dst_ref=ext_ref.at[pl.ds(0, HALO)],
                send_sem=send_sems.at[si],
                recv_sem=recv_sems.at[si],
                device_id=(right,),
                device_id_type=pl.DeviceIdType.MESH,
            )

        def send_left(tref, ext_ref, si):
            return pltpu.make_async_remote_copy(
                src_ref=tref.at[pl.ds(0, HALO)],
                dst_ref=ext_ref.at[pl.ds(EXT - HALO, HALO)],
                send_sem=send_sems.at[si],
                recv_sem=recv_sems.at[si],
                device_id=(left,),
                device_id_type=pl.DeviceIdType.MESH,
            )

        @pl.when(my < N_DEV - 1)
        def _():
            send_right(k_ref, ext_k, 0).start()
            send_right(v_ref, ext_v, 1).start()

        @pl.when(my > 0)
        def _():
            send_left(k_ref, ext_k, 2).start()
            send_left(v_ref, ext_v, 3).start()

        ext_k[pl.ds(HALO, SEQ), :] = k_ref[:, :]
        ext_v[pl.ds(HALO, SEQ), :] = v_ref[:, :]

        zeros = jnp.zeros((HALO, D), jnp.float32)

        @pl.when(my == 0)
        def _():
            ext_k[pl.ds(0, HALO), :] = zeros
            ext_v[pl.ds(0, HALO), :] = zeros

        @pl.when(my == N_DEV - 1)
        def _():
            ext_k[pl.ds(EXT - HALO, HALO), :] = zeros
            ext_v[pl.ds(EXT - HALO, HALO), :] = zeros

        q = jnp.dot(x_ref[:, :], wq_ref[:, :],
                    preferred_element_type=jnp.float32)

        QB = 256
        KW = QB + 2 * HALO
        N_QB = SEQ // QB

        def attn_block(qb):
            r = lax.broadcasted_iota(jnp.int32, (QB, KW), 0)
            c = lax.broadcasted_iota(jnp.int32, (QB, KW), 1)
            kg = my * SEQ - HALO + qb * QB + c
            valid = (jnp.abs(r + HALO - c) <= WINDOW) \
                & (kg >= 0) & (kg < N_DEV * SEQ)
            bias = jnp.where(valid, 0.0, -1e9).astype(jnp.float32)
            for h in range(HQ):
                qh = q[qb * QB:(qb + 1) * QB, h * DH:(h + 1) * DH]
                kh = ext_k[pl.ds(qb * QB, KW), pl.ds(h * DH, DH)]
                vh = ext_v[pl.ds(qb * QB, KW), pl.ds(h * DH, DH)]
                s = lax.dot_general(
                    qh, kh, (((1,), (1,)), ((), ())),
                    preferred_element_type=jnp.float32,
                ) * SCALE + bias
                m = jnp.max(s, axis=1, keepdims=True)
                w = jnp.exp(s - m)
                w = w / jnp.sum(w, axis=1, keepdims=True)
                ctx_ref[pl.ds(qb * QB, QB), pl.ds(h * DH, DH)] = jnp.dot(
                    w, vh, preferred_element_type=jnp.float32)

        attn_block(1)
        attn_block(2)

        @pl.when(my > 0)
        def _():
            send_right(k_ref, ext_k, 0).wait_recv()
            send_right(v_ref, ext_v, 1).wait_recv()

        attn_block(0)

        @pl.when(my < N_DEV - 1)
        def _():
            send_left(k_ref, ext_k, 2).wait_recv()
            send_left(v_ref, ext_v, 3).wait_recv()

        attn_block(N_QB - 1)

        @pl.when(my < N_DEV - 1)
        def _():
            send_right(k_ref, ext_k, 0).wait_send()
            send_right(v_ref, ext_v, 1).wait_send()

        @pl.when(my > 0)
        def _():
            send_left(k_ref, ext_k, 2).wait_send()
            send_left(v_ref, ext_v, 3).wait_send()

        out_ref[:, :] = jnp.dot(ctx_ref[:, :], wo_ref[:, :],
                                preferred_element_type=jnp.float32)

        @functools.partial(pl.run_scoped, sem2=pltpu.SemaphoreType.REGULAR)
        def _(sem2):
            for nbr in (left, right):
                pl.semaphore_signal(sem2, inc=1, device_id=(nbr,),
                                    device_id_type=pl.DeviceIdType.MESH)
            pl.semaphore_wait(sem2, 2)

    out = pl.pallas_call(
        body,
        out_shape=jax.ShapeDtypeStruct((SEQ, D), jnp.float32),
        in_specs=[pl.BlockSpec(memory_space=pltpu.VMEM)] * 5,
        out_specs=pl.BlockSpec(memory_space=pltpu.VMEM),
        scratch_shapes=[
            pltpu.VMEM((EXT, D), jnp.float32),
            pltpu.VMEM((EXT, D), jnp.float32),
            pltpu.VMEM((SEQ, D), jnp.float32),
            pltpu.SemaphoreType.DMA((4,)),
            pltpu.SemaphoreType.DMA((4,)),
        ],
        compiler_params=pltpu.CompilerParams(collective_id=0),
    )(x2, Wq, K2, V2, Wo)
    return out.reshape(1, SEQ, D)


# device time: 31902 ns/iter; 1.5570x vs baseline; 1.0417x over previous
import functools

import jax
import jax.numpy as jnp
from jax import lax
from jax.experimental import pallas as pl
from jax.experimental.pallas import tpu as pltpu

N_DEV = 4
SEQ = 1024
HALO = 128
EXT = SEQ + 2 * HALO
HQ = 8
DH = 128
D = HQ * DH
WINDOW = 128
SCALE = 0.08838834764831843


def kernel(x, Wq, K_ext, V_ext, Wo):
    x2 = x.reshape(SEQ, D)
    K2 = K_ext.reshape(SEQ, D)
    V2 = V_ext.reshape(SEQ, D)

    def body(x_ref, wq_ref, k_ref, v_ref, wo_ref, out_ref,
             ext_k, ext_v, ctx_ref, send_sems, recv_sems):
        my = lax.axis_index("i")
        left = (my - 1) % N_DEV
        right = (my + 1) % N_DEV

        barrier = pltpu.get_barrier_semaphore()
        for nbr in (left, right):
            pl.semaphore_signal(barrier, inc=1, device_id=(nbr,),
                                device_id_type=pl.DeviceIdType.MESH)
        pl.semaphore_wait(barrier, 2)

        def send_right(tref, ext_ref, si):
            return pltpu.make_async_remote_copy(
                src_ref=tref.at[pl.ds(SEQ - HALO, HALO)],
                dst_ref=ext_ref.at[pl.ds(0, HALO)],
                send_sem=send_sems.at[si],
                recv_sem=recv_sems.at[si],
                device_id=(right,),
                device_id_type=pl.DeviceIdType.MESH,
            )

        def send_left(tref, ext_ref, si):
            return pltpu.make_async_remote_copy(
                src_ref=tref.at[pl.ds(0, HALO)],
                dst_ref=ext_ref.at[pl.ds(EXT - HALO, HALO)],
                send_sem=send_sems.at[si],
                recv_sem=recv_sems.at[si],
                device_id=(left,),
                device_id_type=pl.DeviceIdType.MESH,
            )

        @pl.when(my < N_DEV - 1)
        def _():
            send_right(k_ref, ext_k, 0).start()
            send_right(v_ref, ext_v, 1).start()

        @pl.when(my > 0)
        def _():
            send_left(k_ref, ext_k, 2).start()
            send_left(v_ref, ext_v, 3).start()

        ext_k[pl.ds(HALO, SEQ), :] = k_ref[:, :]
        ext_v[pl.ds(HALO, SEQ), :] = v_ref[:, :]

        zeros = jnp.zeros((HALO, D), jnp.float32)

        @pl.when(my == 0)
        def _():
            ext_k[pl.ds(0, HALO), :] = zeros
            ext_v[pl.ds(0, HALO), :] = zeros

        @pl.when(my == N_DEV - 1)
        def _():
            ext_k[pl.ds(EXT - HALO, HALO), :] = zeros
            ext_v[pl.ds(EXT - HALO, HALO), :] = zeros

        q = jnp.dot(x_ref[:, :], wq_ref[:, :],
                    preferred_element_type=jnp.float32) * SCALE

        QB = 256
        KW = QB + 2 * HALO
        N_QB = SEQ // QB

        def attn_block(qb):
            r = lax.broadcasted_iota(jnp.int32, (QB, KW), 0)
            c = lax.broadcasted_iota(jnp.int32, (QB, KW), 1)
            kg = my * SEQ - HALO + qb * QB + c
            valid = (jnp.abs(r + HALO - c) <= WINDOW) \
                & (kg >= 0) & (kg < N_DEV * SEQ)
            bias = jnp.where(valid, 0.0, -1e9).astype(jnp.float32)
            for h in range(HQ):
                qh = q[qb * QB:(qb + 1) * QB, h * DH:(h + 1) * DH]
                kh = ext_k[pl.ds(qb * QB, KW), pl.ds(h * DH, DH)]
                vh = ext_v[pl.ds(qb * QB, KW), pl.ds(h * DH, DH)]
                s = lax.dot_general(
                    qh, kh, (((1,), (1,)), ((), ())),
                    preferred_element_type=jnp.float32,
                ) + bias
                w = jnp.exp(s)
                inv = 1.0 / jnp.sum(w, axis=1, keepdims=True)
                ctx_ref[pl.ds(qb * QB, QB), pl.ds(h * DH, DH)] = jnp.dot(
                    w, vh, preferred_element_type=jnp.float32) * inv

        attn_block(1)
        attn_block(2)

        @pl.when(my > 0)
        def _():
            send_right(k_ref, ext_k, 0).wait_recv()
            send_right(v_ref, ext_v, 1).wait_recv()

        attn_block(0)

        @pl.when(my < N_DEV - 1)
        def _():
            send_left(k_ref, ext_k, 2).wait_recv()
            send_left(v_ref, ext_v, 3).wait_recv()

        attn_block(N_QB - 1)

        @pl.when(my < N_DEV - 1)
        def _():
            send_right(k_ref, ext_k, 0).wait_send()
            send_right(v_ref, ext_v, 1).wait_send()

        @pl.when(my > 0)
        def _():
            send_left(k_ref, ext_k, 2).wait_send()
            send_left(v_ref, ext_v, 3).wait_send()

        out_ref[:, :] = jnp.dot(ctx_ref[:, :], wo_ref[:, :],
                                preferred_element_type=jnp.float32)

        @functools.partial(pl.run_scoped, sem2=pltpu.SemaphoreType.REGULAR)
        def _(sem2):
            for nbr in (left, right):
                pl.semaphore_signal(sem2, inc=1, device_id=(nbr,),
                                    device_id_type=pl.DeviceIdType.MESH)
            pl.semaphore_wait(sem2, 2)

    out = pl.pallas_call(
        body,
        out_shape=jax.ShapeDtypeStruct((SEQ, D), jnp.float32),
        in_specs=[pl.BlockSpec(memory_space=pltpu.VMEM)] * 5,
        out_specs=pl.BlockSpec(memory_space=pltpu.VMEM),
        scratch_shapes=[
            pltpu.VMEM((EXT, D), jnp.float32),
            pltpu.VMEM((EXT, D), jnp.float32),
            pltpu.VMEM((SEQ, D), jnp.float32),
            pltpu.SemaphoreType.DMA((4,)),
            pltpu.SemaphoreType.DMA((4,)),
        ],
        compiler_params=pltpu.CompilerParams(collective_id=0),
    )(x2, Wq, K2, V2, Wo)
    return out.reshape(1, SEQ, D)


# device time: 28279 ns/iter; 1.7565x vs baseline; 1.1281x over previous
import functools

import jax
import jax.numpy as jnp
from jax import lax
from jax.experimental import pallas as pl
from jax.experimental.pallas import tpu as pltpu

N_DEV = 4
SEQ = 1024
HALO = 128
EXT = SEQ + 2 * HALO
HQ = 8
DH = 128
D = HQ * DH
WINDOW = 128
SCALE = 0.08838834764831843


def kernel(x, Wq, K_ext, V_ext, Wo):
    x2 = x.reshape(SEQ, D)
    K2 = K_ext.reshape(SEQ, D)
    V2 = V_ext.reshape(SEQ, D)

    def body(x_ref, wq_ref, k_ref, v_ref, wo_ref, out_ref,
             ext_k, ext_v, ctx_ref, sbuf, send_sems, recv_sems):
        my = lax.axis_index("i")
        left = (my - 1) % N_DEV
        right = (my + 1) % N_DEV

        barrier = pltpu.get_barrier_semaphore()
        for nbr in (left, right):
            pl.semaphore_signal(barrier, inc=1, device_id=(nbr,),
                                device_id_type=pl.DeviceIdType.MESH)
        pl.semaphore_wait(barrier, 2)

        def send_right(si, ext_ref):
            return pltpu.make_async_remote_copy(
                src_ref=sbuf.at[si],
                dst_ref=ext_ref.at[pl.ds(0, HALO)],
                send_sem=send_sems.at[si],
                recv_sem=recv_sems.at[si],
                device_id=(right,),
                device_id_type=pl.DeviceIdType.MESH,
            )

        def send_left(si, ext_ref):
            return pltpu.make_async_remote_copy(
                src_ref=sbuf.at[si],
                dst_ref=ext_ref.at[pl.ds(EXT - HALO, HALO)],
                send_sem=send_sems.at[si],
                recv_sem=recv_sems.at[si],
                device_id=(left,),
                device_id_type=pl.DeviceIdType.MESH,
            )

        @pl.when(my < N_DEV - 1)
        def _():
            sbuf[0, :, :] = k_ref[SEQ - HALO:SEQ, :].astype(jnp.bfloat16)
            sbuf[1, :, :] = v_ref[SEQ - HALO:SEQ, :].astype(jnp.bfloat16)
            send_right(0, ext_k).start()
            send_right(1, ext_v).start()

        @pl.when(my > 0)
        def _():
            sbuf[2, :, :] = k_ref[0:HALO, :].astype(jnp.bfloat16)
            sbuf[3, :, :] = v_ref[0:HALO, :].astype(jnp.bfloat16)
            send_left(2, ext_k).start()
            send_left(3, ext_v).start()

        ext_k[pl.ds(HALO, SEQ), :] = k_ref[:, :].astype(jnp.bfloat16)
        ext_v[pl.ds(HALO, SEQ), :] = v_ref[:, :].astype(jnp.bfloat16)

        zeros = jnp.zeros((HALO, D), jnp.bfloat16)

        @pl.when(my == 0)
        def _():
            ext_k[pl.ds(0, HALO), :] = zeros
            ext_v[pl.ds(0, HALO), :] = zeros

        @pl.when(my == N_DEV - 1)
        def _():
            ext_k[pl.ds(EXT - HALO, HALO), :] = zeros
            ext_v[pl.ds(EXT - HALO, HALO), :] = zeros

        q = (jnp.dot(x_ref[:, :].astype(jnp.bfloat16),
                     wq_ref[:, :].astype(jnp.bfloat16),
                     preferred_element_type=jnp.float32)
             * SCALE).astype(jnp.bfloat16)

        QB = 256
        KW = QB + 2 * HALO
        N_QB = SEQ // QB

        def attn_block(qb):
            r = lax.broadcasted_iota(jnp.int32, (QB, KW), 0)
            c = lax.broadcasted_iota(jnp.int32, (QB, KW), 1)
            kg = my * SEQ - HALO + qb * QB + c
            valid = (jnp.abs(r + HALO - c) <= WINDOW) \
                & (kg >= 0) & (kg < N_DEV * SEQ)
            bias = jnp.where(valid, 0.0, -1e9).astype(jnp.float32)
            for h in range(HQ):
                qh = q[qb * QB:(qb + 1) * QB, h * DH:(h + 1) * DH]
                kh = ext_k[pl.ds(qb * QB, KW), pl.ds(h * DH, DH)]
                vh = ext_v[pl.ds(qb * QB, KW), pl.ds(h * DH, DH)]
                s = lax.dot_general(
                    qh, kh, (((1,), (1,)), ((), ())),
                    preferred_element_type=jnp.float32,
                ) + bias
                w = jnp.exp(s)
                inv = 1.0 / jnp.sum(w, axis=1, keepdims=True)
                ctx_ref[pl.ds(qb * QB, QB), pl.ds(h * DH, DH)] = (jnp.dot(
                    w.astype(jnp.bfloat16), vh,
                    preferred_element_type=jnp.float32) * inv
                ).astype(jnp.bfloat16)

        attn_block(1)
        attn_block(2)

        @pl.when(my > 0)
        def _():
            send_right(0, ext_k).wait_recv()
            send_right(1, ext_v).wait_recv()

        attn_block(0)

        @pl.when(my < N_DEV - 1)
        def _():
            send_left(2, ext_k).wait_recv()
            send_left(3, ext_v).wait_recv()

        attn_block(N_QB - 1)

        @pl.when(my < N_DEV - 1)
        def _():
            send_right(0, ext_k).wait_send()
            send_right(1, ext_v).wait_send()

        @pl.when(my > 0)
        def _():
            send_left(2, ext_k).wait_send()
            send_left(3, ext_v).wait_send()

        out_ref[:, :] = jnp.dot(ctx_ref[:, :],
                                wo_ref[:, :].astype(jnp.bfloat16),
                                preferred_element_type=jnp.float32)

        @functools.partial(pl.run_scoped, sem2=pltpu.SemaphoreType.REGULAR)
        def _(sem2):
            for nbr in (left, right):
                pl.semaphore_signal(sem2, inc=1, device_id=(nbr,),
                                    device_id_type=pl.DeviceIdType.MESH)
            pl.semaphore_wait(sem2, 2)

    out = pl.pallas_call(
        body,
        out_shape=jax.ShapeDtypeStruct((SEQ, D), jnp.float32),
        in_specs=[pl.BlockSpec(memory_space=pltpu.VMEM)] * 5,
        out_specs=pl.BlockSpec(memory_space=pltpu.VMEM),
        scratch_shapes=[
            pltpu.VMEM((EXT, D), jnp.bfloat16),
            pltpu.VMEM((EXT, D), jnp.bfloat16),
            pltpu.VMEM((SEQ, D), jnp.bfloat16),
            pltpu.VMEM((4, HALO, D), jnp.bfloat16),
            pltpu.SemaphoreType.DMA((4,)),
            pltpu.SemaphoreType.DMA((4,)),
        ],
        compiler_params=pltpu.CompilerParams(collective_id=0),
    )(x2, Wq, K2, V2, Wo)
    return out.reshape(1, SEQ, D)
